# baseline (speedup 1.0000x reference)
import functools

import jax
import jax.numpy as jnp
from jax import lax
from jax.experimental import pallas as pl
from jax.experimental.pallas import tpu as pltpu
from jax.experimental.pallas import tpu_sc as plsc

N = 10000
E = 320000
D = 128
H = 128
G = 64

NC = 2
NS = 16
NW = NC * NS
EC = 128
CH0 = 148
CH1 = 12
CR = (CH0 + 1) // 2
Q = 1
E_PAD = NS * (CH0 + CH1) * EC
ZR = 632
N_PAD = NS * ZR

_sc_mesh = plsc.VectorSubcoreMesh(core_axis_name="c", subcore_axis_name="s")


@functools.partial(
    pl.kernel,
    out_type=jax.ShapeDtypeStruct((NC, N_PAD, D), jnp.float32),
    mesh=_sc_mesh,
    scratch_types=[
        pltpu.VMEM((CR, EC), jnp.int32),
        pltpu.VMEM((CR, EC), jnp.int32),
        pltpu.VMEM((Q * EC,), jnp.int32),
        pltpu.VMEM((Q * EC,), jnp.int32),
        pltpu.VMEM((Q * EC, D), jnp.float32),
        pltpu.VMEM_SHARED((N_PAD, D), jnp.float32),
        pltpu.SemaphoreType.DMA,
    ],
)
def _seg_sum_sc(h_hbm, psrc_hbm, pdst_hbm, zeros_hbm, out_hbm,
                psrc_v, pdst_v, sidx2, didx2, rows_v, accum, s0):
    cid = lax.axis_index("c")
    sid = lax.axis_index("s")
    wid = cid * NS + sid

    pltpu.sync_copy(psrc_hbm.at[wid], psrc_v)
    pltpu.sync_copy(pdst_hbm.at[wid], pdst_v)

    def unpack_chunk(packed_ref, out_ref, row, b, base):
        for g in range(4):
            w = packed_ref[row, pl.ds(b * 64 + g * 16, 16)]
            out_ref[pl.ds(base + g * 32, 16)] = w & 0xFFFF
            out_ref[pl.ds(base + g * 32 + 16, 16)] = w >> 16

    for q in range(Q):
        unpack_chunk(psrc_v, sidx2, q // 2, q % 2, q * EC)
        unpack_chunk(pdst_v, didx2, q // 2, q % 2, q * EC)

    pltpu.sync_copy(zeros_hbm, accum.at[pl.ds(sid * ZR, ZR)])
    plsc.subcore_barrier()

    def do_super(g):
        jn = (g + 1) * Q
        pltpu.async_copy(h_hbm.at[sidx2], rows_v, s0).wait()
        for q in range(Q):
            unpack_chunk(psrc_v, sidx2, (jn + q) // 2, (jn + q) % 2, q * EC)
        pltpu.sync_copy(rows_v, accum.at[didx2], add=True)
        for q in range(Q):
            unpack_chunk(pdst_v, didx2, (jn + q) // 2, (jn + q) % 2, q * EC)

    def body(g, carry):
        do_super(g)
        return carry

    n_super = lax.select(cid == 0, CH0 // Q, CH1 // Q)
    lax.fori_loop(0, n_super - 1, body, 0)
    pltpu.async_copy(h_hbm.at[sidx2], rows_v, s0).wait()
    pltpu.sync_copy(rows_v, accum.at[didx2], add=True)
    plsc.subcore_barrier()

    pltpu.sync_copy(accum.at[pl.ds(sid * ZR, ZR)],
                    out_hbm.at[cid, pl.ds(sid * ZR, ZR)])


def _sn_prep_body(ws_ref, out_ref):
    for i in range(6):
        W = ws_ref[i]
        u = jnp.full((1, H), 1.0 / jnp.sqrt(float(H)), jnp.float32)
        for _ in range(3):
            v = jax.lax.dot_general(u, W, (((1,), (0,)), ((), ())),
                                    preferred_element_type=jnp.float32)
            v = v / (jnp.sqrt(jnp.sum(v * v)) + 1e-12)
            u = jax.lax.dot_general(v, W, (((1,), (1,)), ((), ())),
                                    preferred_element_type=jnp.float32)
            u = u / (jnp.sqrt(jnp.sum(u * u)) + 1e-12)
        wv = jax.lax.dot_general(v, W, (((1,), (1,)), ((), ())),
                                 preferred_element_type=jnp.float32)
        sigma = jnp.sum(u * wv)
        out_ref[i] = W / sigma


_sn_prep = pl.pallas_call(
    _sn_prep_body,
    out_shape=jax.ShapeDtypeStruct((6, H, H), jnp.float32),
)

R = 2000


def _mlp_body(h_ref, p0_ref, p1_ref, wa_ref, ba_ref, wb_ref, bb_ref, out_ref):
    z = h_ref[...] + p0_ref[0] + p1_ref[0]
    y = jax.lax.dot_general(z, wa_ref[...], (((1,), (1,)), ((), ())),
                            preferred_element_type=jnp.float32)
    y = jnp.maximum(y + ba_ref[...], 0.0)
    o = jax.lax.dot_general(y, wb_ref[...], (((1,), (1,)), ((), ())),
                            preferred_element_type=jnp.float32)
    out_ref[...] = o + bb_ref[...]


_mlp = pl.pallas_call(
    _mlp_body,
    grid=(N // R,),
    in_specs=[
        pl.BlockSpec((R, D), lambda i: (i, 0)),
        pl.BlockSpec((1, R, D), lambda i: (0, i, 0)),
        pl.BlockSpec((1, R, D), lambda i: (1, i, 0)),
        pl.BlockSpec((H, H), lambda i: (0, 0)),
        pl.BlockSpec((1, H), lambda i: (0, 0)),
        pl.BlockSpec((H, H), lambda i: (0, 0)),
        pl.BlockSpec((1, H), lambda i: (0, 0)),
    ],
    out_specs=pl.BlockSpec((R, D), lambda i: (i, 0)),
    out_shape=jax.ShapeDtypeStruct((N, D), jnp.float32),
)


def _pool_body(h_ref, b_ref, wf_ref, bf_ref, out_ref, sums, counts):
    i = pl.program_id(0)

    @pl.when(i == 0)
    def _():
        sums[...] = jnp.zeros((G, D), jnp.float32)
        counts[...] = jnp.zeros((G, 128), jnp.float32)

    ids = jax.lax.broadcasted_iota(jnp.int32, (G, R), 0)
    m = (ids == b_ref[0, 0][None, :]).astype(jnp.float32)
    sums[...] += jax.lax.dot_general(m, h_ref[...], (((1,), (0,)), ((), ())),
                                     preferred_element_type=jnp.float32)
    counts[...] += jnp.broadcast_to(jnp.sum(m, axis=1, keepdims=True), (G, 128))

    @pl.when(i == pl.num_programs(0) - 1)
    def _():
        pooled = sums[...] / jnp.maximum(counts[...], 1.0)
        s = jnp.sum(pooled * wf_ref[...], axis=1, keepdims=True)
        out_ref[...] = jax.nn.sigmoid(s + bf_ref[0, 0])


_pool = pl.pallas_call(
    _pool_body,
    grid=(N // R,),
    in_specs=[
        pl.BlockSpec((R, D), lambda i: (i, 0)),
        pl.BlockSpec((1, 1, R), lambda i: (i, 0, 0)),
        pl.BlockSpec((1, H), lambda i: (0, 0)),
        pl.BlockSpec((1, 1), lambda i: (0, 0)),
    ],
    out_specs=pl.BlockSpec((G, 1), lambda i: (0, 0)),
    out_shape=jax.ShapeDtypeStruct((G, 1), jnp.float32),
    scratch_shapes=[
        pltpu.VMEM((G, D), jnp.float32),
        pltpu.VMEM((G, 128), jnp.float32),
    ],
)


def kernel(x, edge_index, batch, A0, a0, B0, b0, A1, a1, B1, b1,
           A2, a2, B2, b2, Wf, bf):
    pad = E_PAD - E
    src = jnp.concatenate([edge_index[0], jnp.zeros((pad,), jnp.int32)])
    dst = jnp.concatenate([edge_index[1], jnp.full((pad,), N, jnp.int32)])

    def pack_u16(a):
        def pk(part, ch):
            c = part.reshape(NS, ch, 4, 2, 16)
            return (c[:, :, :, 0, :] | (c[:, :, :, 1, :] << 16)).reshape(
                NS, ch // 2, EC)
        p0 = pk(a[:NS * CH0 * EC], CH0)
        p1 = pk(a[NS * CH0 * EC:], CH1)
        p1 = jnp.concatenate(
            [p1, jnp.zeros((NS, CR - CH1 // 2, EC), jnp.int32)], axis=1)
        return jnp.concatenate([p0, p1], axis=0)

    src_r = pack_u16(src)
    dst_r = pack_u16(dst)
    zeros = jnp.zeros((ZR, D), jnp.float32)

    wn = _sn_prep(jnp.stack([A0, B0, A1, B1, A2, B2]))
    biases = [(a0.reshape(1, H), b0.reshape(1, H)),
              (a1.reshape(1, H), b1.reshape(1, H)),
              (a2.reshape(1, H), b2.reshape(1, H))]

    h = x
    for layer in range(3):
        partials = _seg_sum_sc(h, src_r, dst_r, zeros)
        ba, bb = biases[layer]
        h = _mlp(h, partials, partials, wn[2 * layer], ba,
                 wn[2 * layer + 1], bb)

    batch_r = batch.reshape(N // R, 1, R)
    return _pool(h, batch_r, Wf, bf.reshape(1, 1))


# speedup vs baseline: 1.0548x; 1.0548x over previous
import functools

import jax
import jax.numpy as jnp
from jax import lax
from jax.experimental import pallas as pl
from jax.experimental.pallas import tpu as pltpu
from jax.experimental.pallas import tpu_sc as plsc

N = 10000
E = 320000
D = 128
H = 128
G = 64

NC = 2
NS = 16
NW = NC * NS
EC = 128
CH0 = 144
CH1 = 16
CR = (CH0 + 1) // 2
Q = 1
E_PAD = NS * (CH0 + CH1) * EC
ZR = 632
N_PAD = NS * ZR

_sc_mesh = plsc.VectorSubcoreMesh(core_axis_name="c", subcore_axis_name="s")


@functools.partial(
    pl.kernel,
    out_type=jax.ShapeDtypeStruct((NC, N_PAD, D), jnp.float32),
    mesh=_sc_mesh,
    scratch_types=[
        pltpu.VMEM((CR, EC), jnp.int32),
        pltpu.VMEM((CR, EC), jnp.int32),
        pltpu.VMEM((Q * EC,), jnp.int32),
        pltpu.VMEM((Q * EC,), jnp.int32),
        pltpu.VMEM((Q * EC, D), jnp.float32),
        pltpu.VMEM_SHARED((N_PAD, D), jnp.float32),
        pltpu.SemaphoreType.DMA,
    ],
)
def _seg_sum_sc(h_hbm, psrc_hbm, pdst_hbm, zeros_hbm, out_hbm,
                psrc_v, pdst_v, sidx2, didx2, rows_v, accum, s0):
    cid = lax.axis_index("c")
    sid = lax.axis_index("s")
    wid = cid * NS + sid

    pltpu.sync_copy(psrc_hbm.at[wid], psrc_v)
    pltpu.sync_copy(pdst_hbm.at[wid], pdst_v)

    def unpack_chunk(packed_ref, out_ref, row, b, base):
        for g in range(4):
            w = packed_ref[row, pl.ds(b * 64 + g * 16, 16)]
            out_ref[pl.ds(base + g * 32, 16)] = w & 0xFFFF
            out_ref[pl.ds(base + g * 32 + 16, 16)] = w >> 16

    for q in range(Q):
        unpack_chunk(psrc_v, sidx2, q // 2, q % 2, q * EC)
        unpack_chunk(pdst_v, didx2, q // 2, q % 2, q * EC)

    pltpu.sync_copy(zeros_hbm, accum.at[pl.ds(sid * ZR, ZR)])
    plsc.subcore_barrier()

    def do_super(g):
        jn = (g + 1) * Q
        pltpu.async_copy(h_hbm.at[sidx2], rows_v, s0).wait()
        for q in range(Q):
            unpack_chunk(psrc_v, sidx2, (jn + q) // 2, (jn + q) % 2, q * EC)
        pltpu.sync_copy(rows_v, accum.at[didx2], add=True)
        for q in range(Q):
            unpack_chunk(pdst_v, didx2, (jn + q) // 2, (jn + q) % 2, q * EC)

    def body(g, carry):
        do_super(g)
        return carry

    n_super = lax.select(cid == 0, CH0 // Q, CH1 // Q)
    lax.fori_loop(0, n_super - 1, body, 0)
    pltpu.async_copy(h_hbm.at[sidx2], rows_v, s0).wait()
    pltpu.sync_copy(rows_v, accum.at[didx2], add=True)
    plsc.subcore_barrier()

    pltpu.sync_copy(accum.at[pl.ds(sid * ZR, ZR)],
                    out_hbm.at[cid, pl.ds(sid * ZR, ZR)])


def _sn_prep_body(ws_ref, out_ref):
    for i in range(6):
        W = ws_ref[i]
        u = jnp.full((1, H), 1.0 / jnp.sqrt(float(H)), jnp.float32)
        for _ in range(3):
            v = jax.lax.dot_general(u, W, (((1,), (0,)), ((), ())),
                                    preferred_element_type=jnp.float32)
            v = v / (jnp.sqrt(jnp.sum(v * v)) + 1e-12)
            u = jax.lax.dot_general(v, W, (((1,), (1,)), ((), ())),
                                    preferred_element_type=jnp.float32)
            u = u / (jnp.sqrt(jnp.sum(u * u)) + 1e-12)
        wv = jax.lax.dot_general(v, W, (((1,), (1,)), ((), ())),
                                 preferred_element_type=jnp.float32)
        sigma = jnp.sum(u * wv)
        out_ref[i] = W / sigma


_sn_prep = pl.pallas_call(
    _sn_prep_body,
    out_shape=jax.ShapeDtypeStruct((6, H, H), jnp.float32),
)

R = 2000


def _mlp_body(h_ref, p0_ref, p1_ref, wa_ref, ba_ref, wb_ref, bb_ref, out_ref):
    z = h_ref[...] + p0_ref[0] + p1_ref[0]
    y = jax.lax.dot_general(z, wa_ref[...], (((1,), (1,)), ((), ())),
                            preferred_element_type=jnp.float32)
    y = jnp.maximum(y + ba_ref[...], 0.0)
    o = jax.lax.dot_general(y, wb_ref[...], (((1,), (1,)), ((), ())),
                            preferred_element_type=jnp.float32)
    out_ref[...] = o + bb_ref[...]


_mlp = pl.pallas_call(
    _mlp_body,
    grid=(N // R,),
    in_specs=[
        pl.BlockSpec((R, D), lambda i: (i, 0)),
        pl.BlockSpec((1, R, D), lambda i: (0, i, 0)),
        pl.BlockSpec((1, R, D), lambda i: (1, i, 0)),
        pl.BlockSpec((H, H), lambda i: (0, 0)),
        pl.BlockSpec((1, H), lambda i: (0, 0)),
        pl.BlockSpec((H, H), lambda i: (0, 0)),
        pl.BlockSpec((1, H), lambda i: (0, 0)),
    ],
    out_specs=pl.BlockSpec((R, D), lambda i: (i, 0)),
    out_shape=jax.ShapeDtypeStruct((N, D), jnp.float32),
)


def _pool_body(h_ref, b_ref, wf_ref, bf_ref, out_ref, sums, counts):
    i = pl.program_id(0)

    @pl.when(i == 0)
    def _():
        sums[...] = jnp.zeros((G, D), jnp.float32)
        counts[...] = jnp.zeros((G, 128), jnp.float32)

    ids = jax.lax.broadcasted_iota(jnp.int32, (G, R), 0)
    m = (ids == b_ref[0, 0][None, :]).astype(jnp.float32)
    sums[...] += jax.lax.dot_general(m, h_ref[...], (((1,), (0,)), ((), ())),
                                     preferred_element_type=jnp.float32)
    counts[...] += jnp.broadcast_to(jnp.sum(m, axis=1, keepdims=True), (G, 128))

    @pl.when(i == pl.num_programs(0) - 1)
    def _():
        pooled = sums[...] / jnp.maximum(counts[...], 1.0)
        s = jnp.sum(pooled * wf_ref[...], axis=1, keepdims=True)
        out_ref[...] = jax.nn.sigmoid(s + bf_ref[0, 0])


_pool = pl.pallas_call(
    _pool_body,
    grid=(N // R,),
    in_specs=[
        pl.BlockSpec((R, D), lambda i: (i, 0)),
        pl.BlockSpec((1, 1, R), lambda i: (i, 0, 0)),
        pl.BlockSpec((1, H), lambda i: (0, 0)),
        pl.BlockSpec((1, 1), lambda i: (0, 0)),
    ],
    out_specs=pl.BlockSpec((G, 1), lambda i: (0, 0)),
    out_shape=jax.ShapeDtypeStruct((G, 1), jnp.float32),
    scratch_shapes=[
        pltpu.VMEM((G, D), jnp.float32),
        pltpu.VMEM((G, 128), jnp.float32),
    ],
)


def kernel(x, edge_index, batch, A0, a0, B0, b0, A1, a1, B1, b1,
           A2, a2, B2, b2, Wf, bf):
    pad = E_PAD - E
    src = jnp.concatenate([edge_index[0], jnp.zeros((pad,), jnp.int32)])
    dst = jnp.concatenate([edge_index[1], jnp.full((pad,), N, jnp.int32)])

    def pack_u16(a):
        def pk(part, ch):
            c = part.reshape(NS, ch, 4, 2, 16)
            return (c[:, :, :, 0, :] | (c[:, :, :, 1, :] << 16)).reshape(
                NS, ch // 2, EC)
        p0 = pk(a[:NS * CH0 * EC], CH0)
        p1 = pk(a[NS * CH0 * EC:], CH1)
        p1 = jnp.concatenate(
            [p1, jnp.zeros((NS, CR - CH1 // 2, EC), jnp.int32)], axis=1)
        return jnp.concatenate([p0, p1], axis=0)

    src_r = pack_u16(src)
    dst_r = pack_u16(dst)
    zeros = jnp.zeros((ZR, D), jnp.float32)

    wn = _sn_prep(jnp.stack([A0, B0, A1, B1, A2, B2]))
    biases = [(a0.reshape(1, H), b0.reshape(1, H)),
              (a1.reshape(1, H), b1.reshape(1, H)),
              (a2.reshape(1, H), b2.reshape(1, H))]

    h = x
    for layer in range(3):
        partials = _seg_sum_sc(h, src_r, dst_r, zeros)
        ba, bb = biases[layer]
        h = _mlp(h, partials, partials, wn[2 * layer], ba,
                 wn[2 * layer + 1], bb)

    batch_r = batch.reshape(N // R, 1, R)
    return _pool(h, batch_r, Wf, bf.reshape(1, 1))
